# baseline (speedup 1.0000x reference)
import jax
import jax.numpy as jnp
from jax import lax
from jax.experimental import pallas as pl
from jax.experimental.pallas import tpu as pltpu
from jax.experimental.pallas import tpu_sc as plsc

PATH_NUM = 64
N_TOKENS = 32768
NUM_CORES = 2
NUM_SUBCORES = 16
NUM_WORKERS = NUM_CORES * NUM_SUBCORES
LANES = 16
TGRID = 256
PGRID = 8
CG_PER_WORKER = TGRID // NUM_WORKERS
CGB = 2
NUM_CHUNKS = CG_PER_WORKER // CGB
GROUPS_PER_CHUNK = CGB * 128 // LANES
GB = 4

_NEG_INF = float("-inf")


def _topk_body(x_hbm, o_hbm, vin0, vin1, vout0, vout1,
               isem0, isem1, osem0, osem1):
    wid = lax.axis_index("s") * NUM_CORES + lax.axis_index("c")
    vins = [vin0, vin1]
    vouts = [vout0, vout1]
    isems = [isem0, isem1]
    osems = [osem0, osem1]
    lane = lax.iota(jnp.int32, LANES)
    zero16 = jnp.zeros((LANES,), jnp.int32)
    one16 = jnp.full((LANES,), 1, jnp.int32)
    ninf16 = jnp.full((LANES,), _NEG_INF, jnp.float32)

    def process_group(vin, vout, g):
        cgi = g // (128 // LANES)
        cc0 = (g % (128 // LANES)) * LANES
        m1, i1 = ninf16, zero16
        m2, i2 = ninf16, zero16
        for p in range(PATH_NUM):
            v = vin[p // 8, cgi, p % 8, pl.ds(cc0, LANES)]
            pc = jnp.full((LANES,), p, jnp.int32)
            gt1 = v > m1
            gt2 = v > m2
            m2n = jnp.where(gt2, v, m2)
            i2n = jnp.where(gt2, pc, i2)
            m2 = jnp.where(gt1, m1, m2n)
            i2 = jnp.where(gt1, i1, i2n)
            m1 = jnp.where(gt1, v, m1)
            i1 = jnp.where(gt1, pc, i1)
        cols = lane + cc0
        cg_s = zero16 + cgi
        plsc.store_scatter(vout, [i1 >> 3, cg_s, i1 & 7, cols], one16)
        plsc.store_scatter(vout, [i2 >> 3, cg_s, i2 & 7, cols], one16)

    def make_group_body(vin, vout):
        def group_body(i, carry):
            for gb in range(GB):
                process_group(vin, vout, i * GB + gb)
            return carry
        return group_body

    def zero_chunk(vout):
        def zb(z, carry):
            tg = z // (CGB * 8)
            rem = z % (CGB * 8)
            cgi = rem // 8
            r = rem % 8
            for q in range(128 // LANES):
                vout[tg, cgi, r, pl.ds(q * LANES, LANES)] = zero16
            return carry
        lax.fori_loop(0, PGRID * CGB * 8, zb, 0)

    def hbm_slice(ch):
        cg0 = wid * CG_PER_WORKER + ch * CGB
        return (slice(None), pl.ds(cg0, CGB), slice(None), slice(None))

    out_handles = [None, None]
    pltpu.async_copy(x_hbm.at[hbm_slice(0)], vins[0], isems[0])
    for ch in range(NUM_CHUNKS):
        cur = ch % 2
        if ch + 1 < NUM_CHUNKS:
            nxt = (ch + 1) % 2
            pltpu.async_copy(x_hbm.at[hbm_slice(ch + 1)], vins[nxt],
                             isems[nxt])
        if out_handles[cur] is not None:
            out_handles[cur].wait()
        zero_chunk(vouts[cur])
        pltpu.make_async_copy(x_hbm.at[hbm_slice(ch)], vins[cur],
                              isems[cur]).wait()
        lax.fori_loop(0, GROUPS_PER_CHUNK // GB,
                      make_group_body(vins[cur], vouts[cur]), 0)
        out_handles[cur] = pltpu.async_copy(
            vouts[cur], o_hbm.at[hbm_slice(ch)], osems[cur])
    for h in out_handles:
        if h is not None:
            h.wait()


@jax.jit
def kernel(score):
    mesh = plsc.VectorSubcoreMesh(
        core_axis_name="c", subcore_axis_name="s",
        num_cores=NUM_CORES, num_subcores=NUM_SUBCORES)
    run = pl.kernel(
        _topk_body,
        out_type=jax.ShapeDtypeStruct((PGRID, TGRID, 8, 128), jnp.int32),
        mesh=mesh,
        scratch_types=[
            pltpu.VMEM((PGRID, CGB, 8, 128), jnp.float32),
            pltpu.VMEM((PGRID, CGB, 8, 128), jnp.float32),
            pltpu.VMEM((PGRID, CGB, 8, 128), jnp.int32),
            pltpu.VMEM((PGRID, CGB, 8, 128), jnp.int32),
            pltpu.SemaphoreType.DMA,
            pltpu.SemaphoreType.DMA,
            pltpu.SemaphoreType.DMA,
            pltpu.SemaphoreType.DMA,
        ],
        compiler_params=pltpu.CompilerParams(needs_layout_passes=False),
    )
    x4 = jnp.transpose(jnp.reshape(score, (TGRID, 128, PGRID, 8)),
                       (2, 0, 3, 1))
    o4 = run(x4)
    return jnp.reshape(jnp.transpose(o4, (1, 3, 0, 2)),
                       (N_TOKENS, PATH_NUM))


# speedup vs baseline: 1.2102x; 1.2102x over previous
import jax
import jax.numpy as jnp
from jax import lax
from jax.experimental import pallas as pl
from jax.experimental.pallas import tpu as pltpu
from jax.experimental.pallas import tpu_sc as plsc

PATH_NUM = 64
N_TOKENS = 32768
NUM_CORES = 2
NUM_SUBCORES = 16
NUM_WORKERS = NUM_CORES * NUM_SUBCORES
LANES = 16
TGRID = 256
PGRID = 8
CG_PER_WORKER = TGRID // NUM_WORKERS
CGB = 2
NUM_CHUNKS = CG_PER_WORKER // CGB
GROUPS_PER_CHUNK = CGB * 128 // LANES
GB = 2

_NEG_INF = float("-inf")


def _topk_body(x_hbm, o_hbm, vin0, vin1, vout0, vout1,
               isem0, isem1, osem0, osem1):
    wid = lax.axis_index("s") * NUM_CORES + lax.axis_index("c")
    vins = [vin0, vin1]
    vouts = [vout0, vout1]
    isems = [isem0, isem1]
    osems = [osem0, osem1]
    lane = lax.iota(jnp.int32, LANES)
    zero16 = jnp.zeros((LANES,), jnp.int32)
    one16 = jnp.full((LANES,), 1, jnp.int32)
    ninf16 = jnp.full((LANES,), _NEG_INF, jnp.float32)

    def process_group(vin, vout, g):
        cgi = g // (128 // LANES)
        cc0 = (g % (128 // LANES)) * LANES
        cols = lane + cc0
        cg_s = zero16 + cgi
        m1, q1 = ninf16, zero16
        m2, q2 = ninf16, zero16
        for q in range(PATH_NUM // 2):
            pa, pb = 2 * q, 2 * q + 1
            va = vin[pa // 8, cgi, pa % 8, pl.ds(cc0, LANES)]
            vb = vin[pb // 8, cgi, pb % 8, pl.ds(cc0, LANES)]
            h = jnp.maximum(va, vb)
            qc = jnp.full((LANES,), q, jnp.int32)
            gt1 = h > m1
            gt2 = h > m2
            m2n = jnp.where(gt2, h, m2)
            q2n = jnp.where(gt2, qc, q2)
            m2 = jnp.where(gt1, m1, m2n)
            q2 = jnp.where(gt1, q1, q2n)
            m1 = jnp.where(gt1, h, m1)
            q1 = jnp.where(gt1, qc, q1)
        pa1 = jnp.left_shift(q1, 1)
        d1 = jnp.right_shift(pa1, 3)
        e1 = jnp.bitwise_and(pa1, 7)
        va = plsc.load_gather(vin, [d1, cg_s, e1, cols])
        vb = plsc.load_gather(vin, [d1, cg_s, e1 + 1, cols])
        pa2 = jnp.left_shift(q2, 1)
        d2 = jnp.right_shift(pa2, 3)
        e2 = jnp.bitwise_and(pa2, 7)
        vc = plsc.load_gather(vin, [d2, cg_s, e2, cols])
        vd = plsc.load_gather(vin, [d2, cg_s, e2 + 1, cols])
        lt1 = va < vb
        i1 = jnp.bitwise_or(pa1, jnp.where(lt1, one16, zero16))
        li = jnp.bitwise_or(pa1, jnp.where(lt1, zero16, one16))
        low1 = jnp.minimum(va, vb)
        lt2 = vc < vd
        c2 = jnp.bitwise_or(pa2, jnp.where(lt2, one16, zero16))
        use_l = jnp.logical_or(
            low1 > m2, jnp.logical_and(low1 == m2, li < c2))
        i2 = jnp.where(use_l, li, c2)
        plsc.store_scatter(vout, [i1 >> 3, cg_s, i1 & 7, cols], one16)
        plsc.store_scatter(vout, [i2 >> 3, cg_s, i2 & 7, cols], one16)

    def make_group_body(vin, vout):
        def group_body(i, carry):
            for gb in range(GB):
                process_group(vin, vout, i * GB + gb)
            return carry
        return group_body

    def zero_chunk(vout):
        def zb(z, carry):
            tg = z // (CGB * 8)
            rem = z % (CGB * 8)
            cgi = rem // 8
            r = rem % 8
            for q in range(128 // LANES):
                vout[tg, cgi, r, pl.ds(q * LANES, LANES)] = zero16
            return carry
        lax.fori_loop(0, PGRID * CGB * 8, zb, 0)

    def hbm_slice(ch):
        cg0 = wid * CG_PER_WORKER + ch * CGB
        return (slice(None), pl.ds(cg0, CGB), slice(None), slice(None))

    out_handles = [None, None]
    pltpu.async_copy(x_hbm.at[hbm_slice(0)], vins[0], isems[0])
    for ch in range(NUM_CHUNKS):
        cur = ch % 2
        if ch + 1 < NUM_CHUNKS:
            nxt = (ch + 1) % 2
            pltpu.async_copy(x_hbm.at[hbm_slice(ch + 1)], vins[nxt],
                             isems[nxt])
        if out_handles[cur] is not None:
            out_handles[cur].wait()
        zero_chunk(vouts[cur])
        pltpu.make_async_copy(x_hbm.at[hbm_slice(ch)], vins[cur],
                              isems[cur]).wait()
        lax.fori_loop(0, GROUPS_PER_CHUNK // GB,
                      make_group_body(vins[cur], vouts[cur]), 0)
        out_handles[cur] = pltpu.async_copy(
            vouts[cur], o_hbm.at[hbm_slice(ch)], osems[cur])
    for h in out_handles:
        if h is not None:
            h.wait()


@jax.jit
def kernel(score):
    mesh = plsc.VectorSubcoreMesh(
        core_axis_name="c", subcore_axis_name="s",
        num_cores=NUM_CORES, num_subcores=NUM_SUBCORES)
    run = pl.kernel(
        _topk_body,
        out_type=jax.ShapeDtypeStruct((PGRID, TGRID, 8, 128), jnp.int32),
        mesh=mesh,
        scratch_types=[
            pltpu.VMEM((PGRID, CGB, 8, 128), jnp.float32),
            pltpu.VMEM((PGRID, CGB, 8, 128), jnp.float32),
            pltpu.VMEM((PGRID, CGB, 8, 128), jnp.int32),
            pltpu.VMEM((PGRID, CGB, 8, 128), jnp.int32),
            pltpu.SemaphoreType.DMA,
            pltpu.SemaphoreType.DMA,
            pltpu.SemaphoreType.DMA,
            pltpu.SemaphoreType.DMA,
        ],
        compiler_params=pltpu.CompilerParams(needs_layout_passes=False),
    )
    x4 = jnp.transpose(jnp.reshape(score, (TGRID, 128, PGRID, 8)),
                       (2, 0, 3, 1))
    o4 = run(x4)
    return jnp.reshape(jnp.transpose(o4, (1, 3, 0, 2)),
                       (N_TOKENS, PATH_NUM))


# speedup vs baseline: 1.2625x; 1.0432x over previous
import jax
import jax.numpy as jnp
from jax import lax
from jax.experimental import pallas as pl
from jax.experimental.pallas import tpu as pltpu
from jax.experimental.pallas import tpu_sc as plsc

PATH_NUM = 64
N_TOKENS = 32768
NUM_CORES = 2
NUM_SUBCORES = 16
NUM_WORKERS = NUM_CORES * NUM_SUBCORES
LANES = 16
TGRID = 256
PGRID = 8
CG_PER_WORKER = TGRID // NUM_WORKERS
CGB = 2
NUM_CHUNKS = CG_PER_WORKER // CGB
GROUPS_PER_CHUNK = CGB * 128 // LANES
GB = 1

_NEG_INF = float("-inf")


def _topk_body(x_hbm, o_hbm, vin0, vin1, vout0, vout1,
               isem0, isem1, osem0, osem1):
    wid = lax.axis_index("s") * NUM_CORES + lax.axis_index("c")
    vins = [vin0, vin1]
    vouts = [vout0, vout1]
    isems = [isem0, isem1]
    osems = [osem0, osem1]
    lane = lax.iota(jnp.int32, LANES)
    zero16 = jnp.zeros((LANES,), jnp.int32)
    one16 = jnp.full((LANES,), 1, jnp.int32)
    ninf16 = jnp.full((LANES,), _NEG_INF, jnp.float32)

    def process_group(vin, vout, g):
        cgi = g // (128 // LANES)
        cc0 = (g % (128 // LANES)) * LANES
        cols = lane + cc0
        cg_s = zero16 + cgi
        m1, q1 = ninf16, zero16
        m2, q2 = ninf16, zero16
        for q in range(PATH_NUM // 2):
            pa, pb = 2 * q, 2 * q + 1
            va = vin[pa // 8, cgi, pa % 8, pl.ds(cc0, LANES)]
            vb = vin[pb // 8, cgi, pb % 8, pl.ds(cc0, LANES)]
            h = jnp.maximum(va, vb)
            qc = jnp.full((LANES,), q, jnp.int32)
            gt1 = h > m1
            gt2 = h > m2
            m2n = jnp.where(gt2, h, m2)
            q2n = jnp.where(gt2, qc, q2)
            m2 = jnp.where(gt1, m1, m2n)
            q2 = jnp.where(gt1, q1, q2n)
            m1 = jnp.where(gt1, h, m1)
            q1 = jnp.where(gt1, qc, q1)
        pa1 = jnp.left_shift(q1, 1)
        d1 = jnp.right_shift(pa1, 3)
        e1 = jnp.bitwise_and(pa1, 7)
        va = plsc.load_gather(vin, [d1, cg_s, e1, cols])
        vb = plsc.load_gather(vin, [d1, cg_s, e1 + 1, cols])
        pa2 = jnp.left_shift(q2, 1)
        d2 = jnp.right_shift(pa2, 3)
        e2 = jnp.bitwise_and(pa2, 7)
        vc = plsc.load_gather(vin, [d2, cg_s, e2, cols])
        vd = plsc.load_gather(vin, [d2, cg_s, e2 + 1, cols])
        lt1 = va < vb
        i1 = jnp.bitwise_or(pa1, jnp.where(lt1, one16, zero16))
        li = jnp.bitwise_or(pa1, jnp.where(lt1, zero16, one16))
        low1 = jnp.minimum(va, vb)
        lt2 = vc < vd
        c2 = jnp.bitwise_or(pa2, jnp.where(lt2, one16, zero16))
        use_l = jnp.logical_or(
            low1 > m2, jnp.logical_and(low1 == m2, li < c2))
        i2 = jnp.where(use_l, li, c2)
        plsc.store_scatter(vout, [i1 >> 3, cg_s, i1 & 7, cols], one16)
        plsc.store_scatter(vout, [i2 >> 3, cg_s, i2 & 7, cols], one16)

    def make_group_body(vin, vout):
        def group_body(i, carry):
            for gb in range(GB):
                process_group(vin, vout, i * GB + gb)
            return carry
        return group_body

    def zero_chunk(vout):
        def zb(z, carry):
            tg = z // (CGB * 8)
            rem = z % (CGB * 8)
            cgi = rem // 8
            r = rem % 8
            for q in range(128 // LANES):
                vout[tg, cgi, r, pl.ds(q * LANES, LANES)] = zero16
            return carry
        lax.fori_loop(0, PGRID * CGB * 8, zb, 0)

    def hbm_slice(ch):
        cg0 = wid * CG_PER_WORKER + ch * CGB
        return (slice(None), pl.ds(cg0, CGB), slice(None), slice(None))

    out_handles = [None, None]
    pltpu.async_copy(x_hbm.at[hbm_slice(0)], vins[0], isems[0])
    for ch in range(NUM_CHUNKS):
        cur = ch % 2
        if ch + 1 < NUM_CHUNKS:
            nxt = (ch + 1) % 2
            pltpu.async_copy(x_hbm.at[hbm_slice(ch + 1)], vins[nxt],
                             isems[nxt])
        if out_handles[cur] is not None:
            out_handles[cur].wait()
        zero_chunk(vouts[cur])
        pltpu.make_async_copy(x_hbm.at[hbm_slice(ch)], vins[cur],
                              isems[cur]).wait()
        lax.fori_loop(0, GROUPS_PER_CHUNK // GB,
                      make_group_body(vins[cur], vouts[cur]), 0)
        out_handles[cur] = pltpu.async_copy(
            vouts[cur], o_hbm.at[hbm_slice(ch)], osems[cur])
    for h in out_handles:
        if h is not None:
            h.wait()


@jax.jit
def kernel(score):
    mesh = plsc.VectorSubcoreMesh(
        core_axis_name="c", subcore_axis_name="s",
        num_cores=NUM_CORES, num_subcores=NUM_SUBCORES)
    run = pl.kernel(
        _topk_body,
        out_type=jax.ShapeDtypeStruct((PGRID, TGRID, 8, 128), jnp.int32),
        mesh=mesh,
        scratch_types=[
            pltpu.VMEM((PGRID, CGB, 8, 128), jnp.float32),
            pltpu.VMEM((PGRID, CGB, 8, 128), jnp.float32),
            pltpu.VMEM((PGRID, CGB, 8, 128), jnp.int32),
            pltpu.VMEM((PGRID, CGB, 8, 128), jnp.int32),
            pltpu.SemaphoreType.DMA,
            pltpu.SemaphoreType.DMA,
            pltpu.SemaphoreType.DMA,
            pltpu.SemaphoreType.DMA,
        ],
        compiler_params=pltpu.CompilerParams(needs_layout_passes=False),
    )
    x4 = jnp.transpose(jnp.reshape(score, (TGRID, 128, PGRID, 8)),
                       (2, 0, 3, 1))
    o4 = run(x4)
    return jnp.reshape(jnp.transpose(o4, (1, 3, 0, 2)),
                       (N_TOKENS, PATH_NUM))
